# baseline (speedup 1.0000x reference)
import jax
import jax.numpy as jnp
from jax import lax
from jax.experimental import pallas as pl
from jax.experimental.pallas import tpu as pltpu
from jax.experimental.pallas import tpu_sc as plsc

N = 10000
M = 320000
F = 128
U = 128

NC, NS, L = 2, 16, 16
NW = NC * NS
EPW = M // NW
CH = 80
NCH = EPW // CH
NPAD = 10240
ZCH = NPAD // NS
RPT = NPAD // NS
ZR = 80

_BITREV = (0, 8, 4, 12, 2, 10, 6, 14, 1, 9, 5, 13, 3, 11, 7, 15)


def _shuffle(v, idx):
    return lax.gather(
        v,
        idx[:, None],
        lax.GatherDimensionNumbers(
            offset_dims=(), collapsed_slice_dims=(0,), start_index_map=(0,)
        ),
        (1,),
        mode=lax.GatherScatterMode.PROMISE_IN_BOUNDS,
    )


_mesh = plsc.VectorSubcoreMesh(
    core_axis_name="c", subcore_axis_name="s", num_cores=NC, num_subcores=NS
)


def _tables_body(x_ref, wcat_ref, bcat_ref, wn_ref, sin_ref, sout_ref):
    x = x_ref[...]
    wn_ref[...] = (
        jnp.dot(x, wcat_ref[:, 0:U], preferred_element_type=jnp.float32)
        + bcat_ref[0, :]
    )
    sin_ref[...] = jnp.dot(x, wcat_ref[:, U : 2 * U], preferred_element_type=jnp.float32)
    sout_ref[...] = (
        jnp.dot(x, wcat_ref[:, 2 * U : 3 * U], preferred_element_type=jnp.float32)
        + bcat_ref[1, :]
    )


_TBLK = 2000
_tables = pl.pallas_call(
    _tables_body,
    grid=(N // _TBLK,),
    in_specs=[
        pl.BlockSpec((_TBLK, F), lambda i: (i, 0)),
        pl.BlockSpec((F, 3 * U), lambda i: (0, 0)),
        pl.BlockSpec((2, U), lambda i: (0, 0)),
    ],
    out_specs=[pl.BlockSpec((_TBLK, U), lambda i: (i, 0))] * 3,
    out_shape=[jax.ShapeDtypeStruct((N, U), jnp.float32)] * 3,
)


def _edge_body(
    recvf_h, sendf_h, sin_h, sout_h, wn_h, wal_h,
    pout_h, psum_h,
    ridx0, sidx0, ridx1, sidx1, eav0, eav1,
    rin0, rout0, rin1, rout1, wv, zv, asum_sh, out_sh,
    semG0, semG1, semW0, semW1, semS0, semS1, semC0, semC1,
):
    cid = lax.axis_index("c")
    sid = lax.axis_index("s")
    wid = sid * NC + cid
    pltpu.sync_copy(wal_h, wv)
    lanes = lax.iota(jnp.int32, L)

    def zs(i, c):
        zv[pl.ds(i * L, L)] = jnp.zeros((L,), jnp.float32)
        return c

    lax.fori_loop(0, ZCH // L, zs, 0)
    pltpu.sync_copy(zv, asum_sh.at[pl.ds(sid * ZCH, ZCH)])

    def zr(i, c):
        rin0[i // (U // L), pl.ds((i % (U // L)) * L, L)] = jnp.zeros(
            (L,), jnp.float32
        )
        return c

    lax.fori_loop(0, CH * (U // L), zr, 0)
    for i in range(RPT // ZR):
        pltpu.sync_copy(rin0, out_sh.at[pl.ds(sid * RPT + i * ZR, ZR)])

    def zi(i, c):
        ridx1[pl.ds(i * L, L)] = jnp.zeros((L,), jnp.int32)
        eav1[pl.ds(i * L, L)] = jnp.zeros((L,), jnp.float32)
        return c

    lax.fori_loop(0, CH // L, zi, 0)
    plsc.subcore_barrier()
    pltpu.async_copy(wn_h.at[ridx1], rout1, semW1)
    pltpu.async_copy(eav1, asum_sh.at[ridx1], semC1, add=True)

    base0 = wid * EPW

    def load_idx(t, ridx_b, sidx_b):
        pltpu.sync_copy(recvf_h.at[pl.ds(base0 + t * CH, CH)], ridx_b)
        pltpu.sync_copy(sendf_h.at[pl.ds(base0 + t * CH, CH)], sidx_b)

    def ea_compute(rin, rout, eav_b):
        def edge_group(g, cc):
            accs = []
            for j in _BITREV:
                e = g * L + j
                acc = jnp.zeros((L,), jnp.float32)
                for k in range(U // L):
                    s = pl.ds(k * L, L)
                    z = rin[e, s] + rout[e, s]
                    z = jnp.maximum(z, 0.2 * z)
                    acc = acc + z * wv[s]
                accs.append(acc)
            sh = 8
            while len(accs) > 1:
                nxt = []
                for i in range(0, len(accs), 2):
                    ta = accs[i] + _shuffle(accs[i], lanes ^ sh)
                    tb = accs[i + 1] + _shuffle(accs[i + 1], lanes ^ sh)
                    nxt.append(jnp.where((lanes & sh) == 0, ta, tb))
                accs = nxt
                sh //= 2
            eav_b[pl.ds(g * L, L)] = jnp.exp(accs[0])
            return cc

        lax.fori_loop(0, CH // L, edge_group, 0)

    def scale(eav_b, rows):
        def edge_group(g, cc):
            av = eav_b[pl.ds(g * L, L)]
            for j in range(L):
                a = av[j]
                e = g * L + j
                for k in range(U // L):
                    s = pl.ds(k * L, L)
                    rows[e, s] = rows[e, s] * a
            return cc

        lax.fori_loop(0, CH // L, edge_group, 0)

    load_idx(0, ridx0, sidx0)
    cpi = pltpu.async_copy(sin_h.at[ridx0], rin0, semG0)
    cpo = pltpu.async_copy(sout_h.at[sidx0], rout0, semG0)

    def pair(tt, c):
        a = 2 * tt
        pltpu.make_async_copy(wn_h.at[ridx1], rout1, semW1).wait()
        scale(eav1, rout1)
        pltpu.async_copy(rout1, out_sh.at[ridx1], semS1, add=True)
        pltpu.make_async_copy(sin_h.at[ridx0], rin0, semG0).wait()
        pltpu.make_async_copy(sout_h.at[sidx0], rout0, semG0).wait()
        ea_compute(rin0, rout0, eav0)
        pltpu.async_copy(eav0, asum_sh.at[ridx0], semC0, add=True)
        pltpu.async_copy(wn_h.at[sidx0], rout0, semW0)
        pltpu.make_async_copy(rout1, out_sh.at[ridx1], semS1).wait()
        pltpu.make_async_copy(eav1, asum_sh.at[ridx1], semC1).wait()
        load_idx(a + 1, ridx1, sidx1)
        pltpu.async_copy(sin_h.at[ridx1], rin1, semG1)
        pltpu.async_copy(sout_h.at[sidx1], rout1, semG1)
        pltpu.make_async_copy(wn_h.at[sidx0], rout0, semW0).wait()
        scale(eav0, rout0)
        pltpu.async_copy(rout0, out_sh.at[ridx0], semS0, add=True)
        pltpu.make_async_copy(sin_h.at[ridx1], rin1, semG1).wait()
        pltpu.make_async_copy(sout_h.at[sidx1], rout1, semG1).wait()
        ea_compute(rin1, rout1, eav1)
        pltpu.async_copy(eav1, asum_sh.at[ridx1], semC1, add=True)
        pltpu.async_copy(wn_h.at[sidx1], rout1, semW1)
        pltpu.make_async_copy(rout0, out_sh.at[ridx0], semS0).wait()
        pltpu.make_async_copy(eav0, asum_sh.at[ridx0], semC0).wait()
        load_idx(a + 2, ridx0, sidx0)
        pltpu.async_copy(sin_h.at[ridx0], rin0, semG0)
        pltpu.async_copy(sout_h.at[sidx0], rout0, semG0)
        return c

    lax.fori_loop(0, (NCH - 1) // 2, pair, 0)

    pltpu.make_async_copy(wn_h.at[ridx1], rout1, semW1).wait()
    scale(eav1, rout1)
    pltpu.async_copy(rout1, out_sh.at[ridx1], semS1, add=True)
    pltpu.make_async_copy(sin_h.at[ridx0], rin0, semG0).wait()
    pltpu.make_async_copy(sout_h.at[sidx0], rout0, semG0).wait()
    ea_compute(rin0, rout0, eav0)
    pltpu.async_copy(eav0, asum_sh.at[ridx0], semC0, add=True)
    pltpu.async_copy(wn_h.at[sidx0], rout0, semW0)
    pltpu.make_async_copy(rout1, out_sh.at[ridx1], semS1).wait()
    pltpu.make_async_copy(eav1, asum_sh.at[ridx1], semC1).wait()
    pltpu.make_async_copy(wn_h.at[sidx0], rout0, semW0).wait()
    scale(eav0, rout0)
    pltpu.async_copy(rout0, out_sh.at[ridx0], semS0, add=True)
    pltpu.make_async_copy(rout0, out_sh.at[ridx0], semS0).wait()
    pltpu.make_async_copy(eav0, asum_sh.at[ridx0], semC0).wait()

    plsc.subcore_barrier()

    for i in range(RPT // ZR):
        r = sid * RPT + i * ZR
        pltpu.sync_copy(out_sh.at[pl.ds(r, ZR)], rin0)
        pltpu.sync_copy(rin0, pout_h.at[cid, pl.ds(r, ZR)])

    @pl.when(sid == 0)
    def _():
        pltpu.sync_copy(asum_sh, psum_h.at[pl.ds(cid * NPAD, NPAD)])


_edge_pass = pl.kernel(
    _edge_body,
    out_type=[
        jax.ShapeDtypeStruct((NC, NPAD, U), jnp.float32),
        jax.ShapeDtypeStruct((NC * NPAD,), jnp.float32),
    ],
    mesh=_mesh,
    scratch_types=[
        pltpu.VMEM((CH,), jnp.int32),
        pltpu.VMEM((CH,), jnp.int32),
        pltpu.VMEM((CH,), jnp.int32),
        pltpu.VMEM((CH,), jnp.int32),
        pltpu.VMEM((CH,), jnp.float32),
        pltpu.VMEM((CH,), jnp.float32),
        pltpu.VMEM((CH, U), jnp.float32),
        pltpu.VMEM((CH, U), jnp.float32),
        pltpu.VMEM((CH, U), jnp.float32),
        pltpu.VMEM((CH, U), jnp.float32),
        pltpu.VMEM((U,), jnp.float32),
        pltpu.VMEM((ZCH,), jnp.float32),
        pltpu.VMEM_SHARED((NPAD,), jnp.float32),
        pltpu.VMEM_SHARED((NPAD, U), jnp.float32),
        pltpu.SemaphoreType.DMA,
        pltpu.SemaphoreType.DMA,
        pltpu.SemaphoreType.DMA,
        pltpu.SemaphoreType.DMA,
        pltpu.SemaphoreType.DMA,
        pltpu.SemaphoreType.DMA,
        pltpu.SemaphoreType.DMA,
        pltpu.SemaphoreType.DMA,
    ],
)


def _combine_body(p_ref, s_ref, out_ref):
    inv = 1.0 / jnp.maximum(s_ref[0, :, 0] + s_ref[1, :, 0], 1e-16)
    x = (p_ref[0] + p_ref[1]) * inv[:, None]
    out_ref[...] = jnp.where(x > 0, x, jnp.exp(x) - 1.0)


_combine = pl.pallas_call(
    _combine_body,
    grid=(N // _TBLK,),
    in_specs=[
        pl.BlockSpec((NC, _TBLK, U), lambda i: (0, i, 0)),
        pl.BlockSpec((NC, _TBLK, 1), lambda i: (0, i, 0)),
    ],
    out_specs=pl.BlockSpec((_TBLK, U), lambda i: (i, 0)),
    out_shape=jax.ShapeDtypeStruct((N, U), jnp.float32),
)


def kernel(node, edge, edge_index, W_lin, b_lin, W_att, b_att, w_alpha):
    recvf = edge_index[:, 0]
    sendf = edge_index[:, 1]
    wcat = jnp.concatenate([W_lin, W_att[:F], W_att[F:]], axis=1)
    bcat = jnp.stack([b_lin, b_att])
    wn, sin, sout = _tables(node, wcat, bcat)
    pout, psum = _edge_pass(recvf, sendf, sin, sout, wn, w_alpha[:, 0])
    return _combine(pout[:, :N, :], psum.reshape(NC, NPAD, 1)[:, :N, :])


# speedup vs baseline: 1.4256x; 1.4256x over previous
import jax
import jax.numpy as jnp
from jax import lax
from jax.experimental import pallas as pl
from jax.experimental.pallas import tpu as pltpu
from jax.experimental.pallas import tpu_sc as plsc

N = 10000
M = 320000
F = 128
U = 128

_BITREV = (0, 8, 4, 12, 2, 10, 6, 14, 1, 9, 5, 13, 3, 11, 7, 15)


def _shuffle(v, idx):
    return lax.gather(
        v,
        idx[:, None],
        lax.GatherDimensionNumbers(
            offset_dims=(), collapsed_slice_dims=(0,), start_index_map=(0,)
        ),
        (1,),
        mode=lax.GatherScatterMode.PROMISE_IN_BOUNDS,
    )

NC, NS, L = 2, 16, 16
NW = NC * NS
EPW = M // NW
CH = 80
NCH = EPW // CH
NPAD = 10240
ZCH = NPAD // NS
RPT = NPAD // NS
ZR = 32

_mesh = plsc.VectorSubcoreMesh(
    core_axis_name="c", subcore_axis_name="s", num_cores=NC, num_subcores=NS
)


def _tables_body(x_ref, wcat_ref, bcat_ref, wn_ref, sin_ref, sout_ref):
    x = x_ref[...]
    wn_ref[...] = (
        jnp.dot(x, wcat_ref[:, 0:U], preferred_element_type=jnp.float32)
        + bcat_ref[0, :]
    )
    sin_ref[...] = jnp.dot(x, wcat_ref[:, U : 2 * U], preferred_element_type=jnp.float32)
    sout_ref[...] = (
        jnp.dot(x, wcat_ref[:, 2 * U : 3 * U], preferred_element_type=jnp.float32)
        + bcat_ref[1, :]
    )


_TBLK = 2000
_tables = pl.pallas_call(
    _tables_body,
    grid=(N // _TBLK,),
    in_specs=[
        pl.BlockSpec((_TBLK, F), lambda i: (i, 0)),
        pl.BlockSpec((F, 3 * U), lambda i: (0, 0)),
        pl.BlockSpec((2, U), lambda i: (0, 0)),
    ],
    out_specs=[pl.BlockSpec((_TBLK, U), lambda i: (i, 0))] * 3,
    out_shape=[jax.ShapeDtypeStruct((N, U), jnp.float32)] * 3,
)


def _passA_body(
    recv_h, send_h, sin_h, sout_h, wal_h,
    ea_h, psum_h,
    ridx2, sidx2, rin0, rout0, rin1, rout1, rin2, rout2, eava, wv, zv, asum_sh,
    semA, semB, semC, semD,
):
    cid = lax.axis_index("c")
    sid = lax.axis_index("s")
    wid = sid * NC + cid
    pltpu.sync_copy(wal_h, wv)

    def zbody(i, c):
        zv[pl.ds(i * L, L)] = jnp.zeros((L,), jnp.float32)
        return c

    lax.fori_loop(0, ZCH // L, zbody, 0)
    pltpu.sync_copy(zv, asum_sh.at[pl.ds(sid * ZCH, ZCH)])

    pltpu.sync_copy(recv_h.at[wid], ridx2)
    pltpu.sync_copy(send_h.at[wid], sidx2)
    plsc.subcore_barrier()

    lanes = lax.iota(jnp.int32, L)

    def issue(t, rin_b, rout_b, sem):
        pltpu.async_copy(sin_h.at[ridx2.at[t]], rin_b, sem)
        pltpu.async_copy(sout_h.at[sidx2.at[t]], rout_b, sem)

    def issue_when(t, rin_b, rout_b, sem):
        @pl.when(t < NCH)
        def _():
            issue(t, rin_b, rout_b, sem)

    def drain(rin_b, rout_b, sem):
        pltpu.make_async_copy(sin_h.at[ridx2.at[0]], rin_b, sem).wait()
        pltpu.make_async_copy(sout_h.at[sidx2.at[0]], rout_b, sem).wait()

    def compute(t, rin, rout):
        def edge_group(g, cc):
            accs = []
            for j in _BITREV:
                e = g * L + j
                acc = jnp.zeros((L,), jnp.float32)
                for k in range(U // L):
                    s = pl.ds(k * L, L)
                    z = rin[e, s] + rout[e, s]
                    z = jnp.maximum(z, 0.2 * z)
                    acc = acc + z * wv[s]
                accs.append(acc)
            sh = 8
            while len(accs) > 1:
                nxt = []
                for i in range(0, len(accs), 2):
                    ta = accs[i] + _shuffle(accs[i], lanes ^ sh)
                    tb = accs[i + 1] + _shuffle(accs[i + 1], lanes ^ sh)
                    nxt.append(jnp.where((lanes & sh) == 0, ta, tb))
                accs = nxt
                sh //= 2
            eava[t, pl.ds(g * L, L)] = jnp.exp(accs[0])
            return cc

        lax.fori_loop(0, CH // L, edge_group, 0)
        pltpu.async_copy(eava.at[t], asum_sh.at[ridx2.at[t]], semC, add=True)

    issue(0, rin0, rout0, semA)
    issue(1, rin1, rout1, semB)
    issue(2, rin2, rout2, semD)

    def triple(tt, c):
        a = 3 * tt
        drain(rin0, rout0, semA)
        compute(a, rin0, rout0)
        issue(a + 3, rin0, rout0, semA)
        drain(rin1, rout1, semB)
        compute(a + 1, rin1, rout1)
        issue(a + 4, rin1, rout1, semB)
        drain(rin2, rout2, semD)
        compute(a + 2, rin2, rout2)
        issue_when(a + 5, rin2, rout2, semD)
        return c

    lax.fori_loop(0, (NCH - 2) // 3, triple, 0)
    drain(rin0, rout0, semA)
    compute(NCH - 2, rin0, rout0)
    drain(rin1, rout1, semB)
    compute(NCH - 1, rin1, rout1)

    def drain_sc(t, c):
        pltpu.make_async_copy(
            eava.at[0], asum_sh.at[ridx2.at[0]], semC
        ).wait()
        return c

    lax.fori_loop(0, NCH, drain_sc, 0)
    pltpu.sync_copy(eava, ea_h.at[wid])
    plsc.subcore_barrier()

    @pl.when(sid == 0)
    def _():
        pltpu.sync_copy(asum_sh, psum_h.at[pl.ds(cid * NPAD, NPAD)])


_passA = pl.kernel(
    _passA_body,
    out_type=[
        jax.ShapeDtypeStruct((NW, NCH, CH), jnp.float32),
        jax.ShapeDtypeStruct((NC * NPAD,), jnp.float32),
    ],
    mesh=_mesh,
    scratch_types=[
        pltpu.VMEM((NCH, CH), jnp.int32),
        pltpu.VMEM((NCH, CH), jnp.int32),
        pltpu.VMEM((CH, U), jnp.float32),
        pltpu.VMEM((CH, U), jnp.float32),
        pltpu.VMEM((CH, U), jnp.float32),
        pltpu.VMEM((CH, U), jnp.float32),
        pltpu.VMEM((CH, U), jnp.float32),
        pltpu.VMEM((CH, U), jnp.float32),
        pltpu.VMEM((NCH, CH), jnp.float32),
        pltpu.VMEM((U,), jnp.float32),
        pltpu.VMEM((ZCH,), jnp.float32),
        pltpu.VMEM_SHARED((NPAD,), jnp.float32),
        pltpu.SemaphoreType.DMA,
        pltpu.SemaphoreType.DMA,
        pltpu.SemaphoreType.DMA,
        pltpu.SemaphoreType.DMA,
    ],
)


def _passB_body(
    recvf_h, sendf_h, ea_h, wn_h,
    pout_h,
    ridx0, sidx0, ridx1, sidx1, eava,
    rows0, rows1, zrows, out_sh,
    semA, semB, semS0, semS1,
):
    cid = lax.axis_index("c")
    sid = lax.axis_index("s")
    wid = sid * NC + cid

    def zbody(i, c):
        zrows[i // (U // L), pl.ds((i % (U // L)) * L, L)] = jnp.zeros(
            (L,), jnp.float32
        )
        return c

    lax.fori_loop(0, ZR * (U // L), zbody, 0)
    for i in range(RPT // ZR):
        pltpu.sync_copy(zrows, out_sh.at[pl.ds(sid * RPT + i * ZR, ZR)])

    pltpu.sync_copy(ea_h.at[wid], eava)
    plsc.subcore_barrier()

    base0 = wid * EPW

    def load_idx(t, ridx_b, sidx_b):
        pltpu.sync_copy(recvf_h.at[pl.ds(base0 + t * CH, CH)], ridx_b)
        pltpu.sync_copy(sendf_h.at[pl.ds(base0 + t * CH, CH)], sidx_b)

    def issue(sidx_b, rows_b, sem):
        pltpu.async_copy(wn_h.at[sidx_b], rows_b, sem)

    def drain(sidx_b, rows_b, sem):
        pltpu.make_async_copy(wn_h.at[sidx_b], rows_b, sem).wait()

    def scale(t, rows):
        def edge_group(g, cc):
            av = eava[t, pl.ds(g * L, L)]
            for j in range(L):
                a = av[j]
                e = g * L + j
                for k in range(U // L):
                    s = pl.ds(k * L, L)
                    rows[e, s] = rows[e, s] * a
            return cc

        lax.fori_loop(0, CH // L, edge_group, 0)

    def scat(ridx_b, rows, sem):
        pltpu.async_copy(rows, out_sh.at[ridx_b], sem, add=True)

    def wait_scat(ridx_b, rows, sem):
        pltpu.make_async_copy(rows, out_sh.at[ridx_b], sem).wait()

    def zb2(i, c):
        rows1[i // (U // L), pl.ds((i % (U // L)) * L, L)] = jnp.zeros(
            (L,), jnp.float32
        )
        return c

    lax.fori_loop(0, CH * (U // L), zb2, 0)

    def zi(i, c):
        ridx1[pl.ds(i * L, L)] = jnp.zeros((L,), jnp.int32)
        return c

    lax.fori_loop(0, CH // L, zi, 0)
    scat(ridx1, rows1, semS1)

    load_idx(0, ridx0, sidx0)
    issue(sidx0, rows0, semA)

    def pair(tt, c):
        a = 2 * tt
        wait_scat(ridx1, rows1, semS1)
        load_idx(a + 1, ridx1, sidx1)
        issue(sidx1, rows1, semB)
        drain(sidx0, rows0, semA)
        scale(a, rows0)
        scat(ridx0, rows0, semS0)
        drain(sidx1, rows1, semB)
        scale(a + 1, rows1)
        scat(ridx1, rows1, semS1)
        wait_scat(ridx0, rows0, semS0)
        load_idx(a + 2, ridx0, sidx0)
        issue(sidx0, rows0, semA)
        return c

    lax.fori_loop(0, (NCH - 1) // 2, pair, 0)
    wait_scat(ridx1, rows1, semS1)
    drain(sidx0, rows0, semA)
    scale(NCH - 1, rows0)
    scat(ridx0, rows0, semS0)
    wait_scat(ridx0, rows0, semS0)

    plsc.subcore_barrier()

    for blk in range(RPT // ZR):
        r = sid * RPT + blk * ZR
        pltpu.sync_copy(out_sh.at[pl.ds(r, ZR)], pout_h.at[cid, pl.ds(r, ZR)])


_passB = pl.kernel(
    _passB_body,
    out_type=jax.ShapeDtypeStruct((NC, NPAD, U), jnp.float32),
    mesh=_mesh,
    scratch_types=[
        pltpu.VMEM((CH,), jnp.int32),
        pltpu.VMEM((CH,), jnp.int32),
        pltpu.VMEM((CH,), jnp.int32),
        pltpu.VMEM((CH,), jnp.int32),
        pltpu.VMEM((NCH, CH), jnp.float32),
        pltpu.VMEM((CH, U), jnp.float32),
        pltpu.VMEM((CH, U), jnp.float32),
        pltpu.VMEM((ZR, U), jnp.float32),
        pltpu.VMEM_SHARED((NPAD, U), jnp.float32),
        pltpu.SemaphoreType.DMA,
        pltpu.SemaphoreType.DMA,
        pltpu.SemaphoreType.DMA,
        pltpu.SemaphoreType.DMA,
    ],
)


def _combine_body(p_ref, s_ref, out_ref):
    inv = 1.0 / jnp.maximum(s_ref[0, :, 0] + s_ref[1, :, 0], 1e-16)
    x = (p_ref[0] + p_ref[1]) * inv[:, None]
    out_ref[...] = jnp.where(x > 0, x, jnp.exp(x) - 1.0)


_combine = pl.pallas_call(
    _combine_body,
    grid=(N // _TBLK,),
    in_specs=[
        pl.BlockSpec((NC, _TBLK, U), lambda i: (0, i, 0)),
        pl.BlockSpec((NC, _TBLK, 1), lambda i: (0, i, 0)),
    ],
    out_specs=pl.BlockSpec((_TBLK, U), lambda i: (i, 0)),
    out_shape=jax.ShapeDtypeStruct((N, U), jnp.float32),
)


def kernel(node, edge, edge_index, W_lin, b_lin, W_att, b_att, w_alpha):
    recvf = edge_index[:, 0]
    sendf = edge_index[:, 1]
    recv3 = recvf.reshape(NW, NCH, CH)
    send3 = sendf.reshape(NW, NCH, CH)
    wcat = jnp.concatenate([W_lin, W_att[:F], W_att[F:]], axis=1)
    bcat = jnp.stack([b_lin, b_att])
    wn, sin, sout = _tables(node, wcat, bcat)
    ea, psum = _passA(recv3, send3, sin, sout, w_alpha[:, 0])
    pout = _passB(recvf, sendf, ea, wn)
    return _combine(pout[:, :N, :], psum.reshape(NC, NPAD, 1)[:, :N, :])


# speedup vs baseline: 1.5451x; 1.0838x over previous
import jax
import jax.numpy as jnp
from jax import lax
from jax.experimental import pallas as pl
from jax.experimental.pallas import tpu as pltpu
from jax.experimental.pallas import tpu_sc as plsc

N = 10000
M = 320000
F = 128
U = 128

_BITREV = (0, 8, 4, 12, 2, 10, 6, 14, 1, 9, 5, 13, 3, 11, 7, 15)


def _shuffle(v, idx):
    return lax.gather(
        v,
        idx[:, None],
        lax.GatherDimensionNumbers(
            offset_dims=(), collapsed_slice_dims=(0,), start_index_map=(0,)
        ),
        (1,),
        mode=lax.GatherScatterMode.PROMISE_IN_BOUNDS,
    )

NC, NS, L = 2, 16, 16
NW = NC * NS
EPW = M // NW
CH = 80
NCH = EPW // CH
NPAD = 10240
ZCH = NPAD // NS
RPT = NPAD // NS
ZR = 32

_mesh = plsc.VectorSubcoreMesh(
    core_axis_name="c", subcore_axis_name="s", num_cores=NC, num_subcores=NS
)


def _tables_body(x_ref, wcat_ref, bcat_ref, wn_ref, sin_ref, sout_ref):
    x = x_ref[...]
    wn_ref[...] = (
        jnp.dot(x, wcat_ref[:, 0:U], preferred_element_type=jnp.float32)
        + bcat_ref[0, :]
    )
    sin_ref[...] = jnp.dot(x, wcat_ref[:, U : 2 * U], preferred_element_type=jnp.float32)
    sout_ref[...] = (
        jnp.dot(x, wcat_ref[:, 2 * U : 3 * U], preferred_element_type=jnp.float32)
        + bcat_ref[1, :]
    )


_TBLK = 2000
_tables = pl.pallas_call(
    _tables_body,
    grid=(N // _TBLK,),
    in_specs=[
        pl.BlockSpec((_TBLK, F), lambda i: (i, 0)),
        pl.BlockSpec((F, 3 * U), lambda i: (0, 0)),
        pl.BlockSpec((2, U), lambda i: (0, 0)),
    ],
    out_specs=[pl.BlockSpec((_TBLK, U), lambda i: (i, 0))] * 3,
    out_shape=[jax.ShapeDtypeStruct((N, U), jnp.float32)] * 3,
)


def _passA_body(
    recv_h, send_h, sin_h, sout_h, wal_h,
    ea_h, psum_h,
    ridx2, sidx2, rin0, rout0, rin1, rout1, rin2, rout2, eava, wv, zv, asum_sh,
    semA, semB, semC, semD,
):
    cid = lax.axis_index("c")
    sid = lax.axis_index("s")
    wid = sid * NC + cid
    pltpu.sync_copy(wal_h, wv)

    def zbody(i, c):
        zv[pl.ds(i * L, L)] = jnp.zeros((L,), jnp.float32)
        return c

    lax.fori_loop(0, ZCH // L, zbody, 0)
    pltpu.sync_copy(zv, asum_sh.at[pl.ds(sid * ZCH, ZCH)])

    pltpu.sync_copy(recv_h.at[wid], ridx2)
    pltpu.sync_copy(send_h.at[wid], sidx2)
    plsc.subcore_barrier()

    lanes = lax.iota(jnp.int32, L)

    def issue(t, rin_b, rout_b, sem):
        pltpu.async_copy(sin_h.at[ridx2.at[t]], rin_b, sem)
        pltpu.async_copy(sout_h.at[sidx2.at[t]], rout_b, sem)

    def issue_when(t, rin_b, rout_b, sem):
        @pl.when(t < NCH)
        def _():
            issue(t, rin_b, rout_b, sem)

    def drain(rin_b, rout_b, sem):
        pltpu.make_async_copy(sin_h.at[ridx2.at[0]], rin_b, sem).wait()
        pltpu.make_async_copy(sout_h.at[sidx2.at[0]], rout_b, sem).wait()

    def compute(t, rin, rout):
        def edge_group(g, cc):
            accs = []
            for j in _BITREV:
                e = g * L + j
                acc = jnp.zeros((L,), jnp.float32)
                for k in range(U // L):
                    s = pl.ds(k * L, L)
                    z = rin[e, s] + rout[e, s]
                    z = jnp.maximum(z, 0.2 * z)
                    acc = acc + z * wv[s]
                accs.append(acc)
            sh = 8
            while len(accs) > 1:
                nxt = []
                for i in range(0, len(accs), 2):
                    ta = accs[i] + _shuffle(accs[i], lanes ^ sh)
                    tb = accs[i + 1] + _shuffle(accs[i + 1], lanes ^ sh)
                    nxt.append(jnp.where((lanes & sh) == 0, ta, tb))
                accs = nxt
                sh //= 2
            eava[t, pl.ds(g * L, L)] = jnp.exp(accs[0])
            return cc

        lax.fori_loop(0, CH // L, edge_group, 0)
        pltpu.async_copy(eava.at[t], asum_sh.at[ridx2.at[t]], semC, add=True)

    issue(0, rin0, rout0, semA)
    issue(1, rin1, rout1, semB)
    issue(2, rin2, rout2, semD)

    def triple(tt, c):
        a = 3 * tt
        drain(rin0, rout0, semA)
        compute(a, rin0, rout0)
        issue(a + 3, rin0, rout0, semA)
        drain(rin1, rout1, semB)
        compute(a + 1, rin1, rout1)
        issue(a + 4, rin1, rout1, semB)
        drain(rin2, rout2, semD)
        compute(a + 2, rin2, rout2)
        issue_when(a + 5, rin2, rout2, semD)
        return c

    lax.fori_loop(0, (NCH - 2) // 3, triple, 0)
    drain(rin0, rout0, semA)
    compute(NCH - 2, rin0, rout0)
    drain(rin1, rout1, semB)
    compute(NCH - 1, rin1, rout1)

    def drain_sc(t, c):
        pltpu.make_async_copy(
            eava.at[0], asum_sh.at[ridx2.at[0]], semC
        ).wait()
        return c

    lax.fori_loop(0, NCH, drain_sc, 0)
    pltpu.sync_copy(eava, ea_h.at[wid])
    plsc.subcore_barrier()

    @pl.when(sid == 0)
    def _():
        pltpu.sync_copy(asum_sh, psum_h.at[pl.ds(cid * NPAD, NPAD)])


_passA = pl.kernel(
    _passA_body,
    out_type=[
        jax.ShapeDtypeStruct((NW, NCH, CH), jnp.float32),
        jax.ShapeDtypeStruct((NC * NPAD,), jnp.float32),
    ],
    mesh=_mesh,
    scratch_types=[
        pltpu.VMEM((NCH, CH), jnp.int32),
        pltpu.VMEM((NCH, CH), jnp.int32),
        pltpu.VMEM((CH, U), jnp.float32),
        pltpu.VMEM((CH, U), jnp.float32),
        pltpu.VMEM((CH, U), jnp.float32),
        pltpu.VMEM((CH, U), jnp.float32),
        pltpu.VMEM((CH, U), jnp.float32),
        pltpu.VMEM((CH, U), jnp.float32),
        pltpu.VMEM((NCH, CH), jnp.float32),
        pltpu.VMEM((U,), jnp.float32),
        pltpu.VMEM((ZCH,), jnp.float32),
        pltpu.VMEM_SHARED((NPAD,), jnp.float32),
        pltpu.SemaphoreType.DMA,
        pltpu.SemaphoreType.DMA,
        pltpu.SemaphoreType.DMA,
        pltpu.SemaphoreType.DMA,
    ],
)


def _passB_body(
    recvf_h, sendf_h, ea_h, wn_h,
    pout_h,
    ridx0, sidx0, ridx1, sidx1, ridx2, sidx2, eava,
    rows0, rows1, rows2, out_sh,
    semG0, semG1, semG2, semS0, semS1, semS2,
):
    cid = lax.axis_index("c")
    sid = lax.axis_index("s")
    wid = sid * NC + cid

    rows = (rows0, rows1, rows2)
    ridx = (ridx0, ridx1, ridx2)
    sidx = (sidx0, sidx1, sidx2)
    semG = (semG0, semG1, semG2)
    semS = (semS0, semS1, semS2)

    def zbody(i, c):
        rows0[i // (U // L), pl.ds((i % (U // L)) * L, L)] = jnp.zeros(
            (L,), jnp.float32
        )
        return c

    lax.fori_loop(0, CH * (U // L), zbody, 0)
    for i in range(RPT // CH):
        pltpu.sync_copy(rows0, out_sh.at[pl.ds(sid * RPT + i * CH, CH)])

    def zb2(i, c):
        rows2[i // (U // L), pl.ds((i % (U // L)) * L, L)] = jnp.zeros(
            (L,), jnp.float32
        )
        return c

    lax.fori_loop(0, CH * (U // L), zb2, 0)

    def zi(i, c):
        ridx2[pl.ds(i * L, L)] = jnp.zeros((L,), jnp.int32)
        return c

    lax.fori_loop(0, CH // L, zi, 0)

    pltpu.sync_copy(ea_h.at[wid], eava)
    plsc.subcore_barrier()

    base0 = wid * EPW

    def load_idx(t, b):
        pltpu.sync_copy(recvf_h.at[pl.ds(base0 + t * CH, CH)], ridx[b])
        pltpu.sync_copy(sendf_h.at[pl.ds(base0 + t * CH, CH)], sidx[b])

    def issueG(b):
        pltpu.async_copy(wn_h.at[sidx[b]], rows[b], semG[b])

    def drainG(b):
        pltpu.make_async_copy(wn_h.at[sidx[b]], rows[b], semG[b]).wait()

    def scale(t, rows_b):
        def edge_group(g, cc):
            av = eava[t, pl.ds(g * L, L)]
            for j in range(L):
                a = av[j]
                e = g * L + j
                for k in range(U // L):
                    s = pl.ds(k * L, L)
                    rows_b[e, s] = rows_b[e, s] * a
            return cc

        lax.fori_loop(0, CH // L, edge_group, 0)

    def scat(b):
        pltpu.async_copy(rows[b], out_sh.at[ridx[b]], semS[b], add=True)

    def wait_scat(b):
        pltpu.make_async_copy(rows[b], out_sh.at[ridx[b]], semS[b]).wait()

    def slot(c, b, do_issue=True):
        bb = (b + 2) % 3
        drainG(b)
        scale(c, rows[b])
        wait_scat(bb)
        if do_issue:
            load_idx(c + 2, bb)
            issueG(bb)
        scat(b)

    scat(2)
    load_idx(0, 0)
    issueG(0)
    load_idx(1, 1)
    issueG(1)

    def triple(i, c):
        a = 3 * i
        slot(a, 0)
        slot(a + 1, 1)
        slot(a + 2, 2)
        return c

    lax.fori_loop(0, (NCH - 2) // 3, triple, 0)
    slot(NCH - 2, 0, do_issue=False)
    slot(NCH - 1, 1, do_issue=False)
    wait_scat(1)

    plsc.subcore_barrier()

    for blk in range(RPT // CH):
        r = sid * RPT + blk * CH
        pltpu.sync_copy(out_sh.at[pl.ds(r, CH)], pout_h.at[cid, pl.ds(r, CH)])


_passB = pl.kernel(
    _passB_body,
    out_type=jax.ShapeDtypeStruct((NC, NPAD, U), jnp.float32),
    mesh=_mesh,
    scratch_types=[
        pltpu.VMEM((CH,), jnp.int32),
        pltpu.VMEM((CH,), jnp.int32),
        pltpu.VMEM((CH,), jnp.int32),
        pltpu.VMEM((CH,), jnp.int32),
        pltpu.VMEM((CH,), jnp.int32),
        pltpu.VMEM((CH,), jnp.int32),
        pltpu.VMEM((NCH, CH), jnp.float32),
        pltpu.VMEM((CH, U), jnp.float32),
        pltpu.VMEM((CH, U), jnp.float32),
        pltpu.VMEM((CH, U), jnp.float32),
        pltpu.VMEM_SHARED((NPAD, U), jnp.float32),
        pltpu.SemaphoreType.DMA,
        pltpu.SemaphoreType.DMA,
        pltpu.SemaphoreType.DMA,
        pltpu.SemaphoreType.DMA,
        pltpu.SemaphoreType.DMA,
        pltpu.SemaphoreType.DMA,
    ],
)


def _combine_body(p_ref, s_ref, out_ref):
    inv = 1.0 / jnp.maximum(s_ref[0, :, 0] + s_ref[1, :, 0], 1e-16)
    x = (p_ref[0] + p_ref[1]) * inv[:, None]
    out_ref[...] = jnp.where(x > 0, x, jnp.exp(x) - 1.0)


_combine = pl.pallas_call(
    _combine_body,
    grid=(N // _TBLK,),
    in_specs=[
        pl.BlockSpec((NC, _TBLK, U), lambda i: (0, i, 0)),
        pl.BlockSpec((NC, _TBLK, 1), lambda i: (0, i, 0)),
    ],
    out_specs=pl.BlockSpec((_TBLK, U), lambda i: (i, 0)),
    out_shape=jax.ShapeDtypeStruct((N, U), jnp.float32),
)


def kernel(node, edge, edge_index, W_lin, b_lin, W_att, b_att, w_alpha):
    recvf = edge_index[:, 0]
    sendf = edge_index[:, 1]
    recv3 = recvf.reshape(NW, NCH, CH)
    send3 = sendf.reshape(NW, NCH, CH)
    wcat = jnp.concatenate([W_lin, W_att[:F], W_att[F:]], axis=1)
    bcat = jnp.stack([b_lin, b_att])
    wn, sin, sout = _tables(node, wcat, bcat)
    ea, psum = _passA(recv3, send3, sin, sout, w_alpha[:, 0])
    pout = _passB(recvf, sendf, ea, wn)
    return _combine(pout[:, :N, :], psum.reshape(NC, NPAD, 1)[:, :N, :])
